# baseline (speedup 1.0000x reference)
import jax
import jax.numpy as jnp
from jax.experimental import pallas as pl

HEIGHT = 128
MAX_SCALE = 3.5


def _onehot_kernel(x_ref, o_ref):
    x = x_ref[...]
    xc = jnp.clip(x, -MAX_SCALE, MAX_SCALE)
    bins = (xc + MAX_SCALE) / (2.0 * MAX_SCALE) * HEIGHT
    idx = jnp.clip(bins.astype(jnp.int32), 0, HEIGHT - 1)
    bb, t = x.shape
    rows = jax.lax.broadcasted_iota(jnp.int32, (bb, 1, HEIGHT, t), 2)
    o_ref[...] = (rows == idx[:, None, None, :]).astype(jnp.float32)


def kernel(x):
    batch, seq = x.shape
    bb = 8
    return pl.pallas_call(
        _onehot_kernel,
        grid=(batch // bb,),
        in_specs=[pl.BlockSpec((bb, seq), lambda i: (i, 0))],
        out_specs=pl.BlockSpec((bb, 1, HEIGHT, seq), lambda i: (i, 0, 0, 0)),
        out_shape=jax.ShapeDtypeStruct((batch, 1, HEIGHT, seq), jnp.float32),
    )(x)
